# baseline (speedup 1.0000x reference)
import functools

import jax
import jax.numpy as jnp
from jax import lax
from jax.experimental import pallas as pl
from jax.experimental.pallas import tpu as pltpu
from jax.experimental.pallas import tpu_sc as plsc

NC = 2
NS = 16
NW = NC * NS
CHUNK = 128
NBUF = 5
PRIME = 4
LAG = NBUF - PRIME
PAD = 128


def _mesh():
    return plsc.VectorSubcoreMesh(
        core_axis_name="c", subcore_axis_name="s",
        num_cores=NC, num_subcores=NS)


def _wid():
    return lax.axis_index("s") * NC + lax.axis_index("c")


def _build_transpose(emb_dim: int, n_full_cols: int, out_rows: int):
    per_w = n_full_cols // NW
    rem = n_full_cols - per_w * NW
    assert per_w % 2 == 0 and per_w >= 4

    @functools.partial(
        pl.kernel,
        out_type=jax.ShapeDtypeStruct((out_rows, PAD), jnp.float32),
        mesh=_mesh(),
        scratch_types=[
            pltpu.VMEM((2, emb_dim, 128), jnp.float32),
            pltpu.VMEM((2, 128, PAD), jnp.float32),
            pltpu.SemaphoreType.DMA,
            pltpu.SemaphoreType.DMA,
        ],
        compiler_params=pltpu.CompilerParams(use_tc_tiling_on_sc=True, needs_layout_passes=False),
    )
    def transpose_k(tab_t_hbm, tail_hbm, out_hbm, xb, yb, sem_i, sem_o):
        wid = _wid()
        base = wid * per_w

        def in_desc(tc, b):
            return pltpu.make_async_copy(
                tab_t_hbm.at[:, pl.ds(tc * 128, 128)], xb.at[b], sem_i)

        def out_desc(tc, b):
            return pltpu.make_async_copy(
                yb.at[b], out_hbm.at[pl.ds(tc * 128, 128)], sem_o)

        lanes = lax.iota(jnp.int32, 16)

        def transpose_buf(b):
            def tbody(l, carry):
                lsplat = jnp.full((16,), l, jnp.int32)
                for k in range(emb_dim // 16):
                    vec = plsc.load_gather(
                        xb, [jnp.full((16,), b, jnp.int32),
                             lanes + 16 * k, lsplat])
                    yb[b, l, pl.ds(16 * k, 16)] = vec
                return carry
            lax.fori_loop(0, 128, tbody, 0)

        def step(i, b, first, last):
            tc = base + 2 * i + b
            in_desc(tc, b).wait()
            if not first:
                out_desc(tc - 2, b).wait()
            transpose_buf(b)
            out_desc(tc, b).start()
            if not last:
                in_desc(tc + 2, b).start()

        in_desc(base, 0).start()
        in_desc(base + 1, 1).start()
        for b in range(2):
            step(0, b, True, False)

        def middle(i, carry):
            for b in range(2):
                step(i, b, False, False)
            return carry

        lax.fori_loop(1, per_w // 2 - 1, middle, 0)

        for b in range(2):
            step(per_w // 2 - 1, b, False, True)
        for b in range(2):
            out_desc(base + per_w - 2 + b, b).wait()

        @pl.when(wid < rem)
        def _():
            tc = NW * per_w + wid
            pltpu.sync_copy(tab_t_hbm.at[:, pl.ds(tc * 128, 128)], xb.at[0])
            transpose_buf(0)
            pltpu.sync_copy(yb.at[0], out_hbm.at[pl.ds(tc * 128, 128)])

        @pl.when(wid == NW - 1)
        def _():
            pltpu.sync_copy(tail_hbm, xb.at[0, :, :])
            pltpu.sync_copy(
                xb.at[0, :, :],
                out_hbm.at[pl.ds(n_full_cols * 128, emb_dim)])

    return transpose_k


def _build_gather(total: int, table_rows: int):
    n_chunks = total // (NW * CHUNK)
    assert n_chunks * NW * CHUNK == total
    assert n_chunks % NBUF == 0 and n_chunks >= 2 * NBUF
    n_outer = n_chunks // NBUF
    rows_per_worker = n_chunks * CHUNK

    @functools.partial(
        pl.kernel,
        out_type=jax.ShapeDtypeStruct((total, PAD), jnp.float32),
        mesh=_mesh(),
        scratch_types=[
            pltpu.VMEM((n_chunks, CHUNK), jnp.int32),
            pltpu.VMEM((NBUF, CHUNK, PAD), jnp.float32),
            pltpu.SemaphoreType.DMA,
            pltpu.SemaphoreType.DMA,
        ],
        compiler_params=pltpu.CompilerParams(use_tc_tiling_on_sc=True),
    )
    def emb_gather(table_hbm, idx_hbm, out_hbm, idx_v, bufs, sem_g, sem_o):
        wid = _wid()
        base = wid * rows_per_worker
        pltpu.sync_copy(idx_hbm.at[wid], idx_v)

        def gather_desc(g, b):
            return pltpu.make_async_copy(table_hbm.at[idx_v.at[g]],
                                         bufs.at[b], sem_g)

        def scatter_desc(g, b):
            return pltpu.make_async_copy(
                bufs.at[b],
                out_hbm.at[pl.ds(base + g * CHUNK, CHUNK)], sem_o)

        for b in range(PRIME):
            gather_desc(b, b).start()

        def step(t, b, first, last):
            gather_desc(t, b).wait()
            scatter_desc(t, b).start()
            if first and b < LAG:
                gather_desc(t + PRIME, (b + PRIME) % NBUF).start()
            elif not (last and b >= LAG):
                scatter_desc(t - LAG, (b - LAG) % NBUF).wait()
                gather_desc(t + PRIME, (b + PRIME) % NBUF).start()
            else:
                scatter_desc(t - LAG, (b - LAG) % NBUF).wait()

        for b in range(NBUF):
            step(b, b, True, False)

        def outer(i, carry):
            t0 = i * NBUF
            for b in range(NBUF):
                step(t0 + b, b, False, False)
            return carry

        lax.fori_loop(1, n_outer - 1, outer, 0)

        for b in range(NBUF):
            step(n_chunks - NBUF + b, b, False, True)

        for k in range(LAG):
            g = n_chunks - LAG + k
            scatter_desc(g, g % NBUF).wait()

    return emb_gather


def kernel(batch_input, lengths, embedding_table):
    del lengths
    batch, hist = batch_input.shape
    vocab, emb_dim = embedding_table.shape
    total = batch * hist
    n_chunks = total // (NW * CHUNK)
    idx = batch_input.reshape(NW, n_chunks, CHUNK).astype(jnp.int32)

    n_full_cols = vocab // 128
    v_main = n_full_cols * 128
    out_rows = (n_full_cols + 1) * 128

    tab_t = embedding_table.T
    tail = jnp.pad(embedding_table[v_main:, :],
                   ((0, 0), (0, PAD - emb_dim)))

    table_pad = _build_transpose(emb_dim, n_full_cols, out_rows)(tab_t, tail)
    out = _build_gather(total, out_rows)(table_pad, idx)
    return out[:, :emb_dim].reshape(batch, hist, emb_dim)
